# baseline (speedup 1.0000x reference)
import functools

import jax
import jax.numpy as jnp
from jax import lax
from jax.experimental import pallas as pl
from jax.experimental.pallas import tpu as pltpu
from jax.experimental.pallas import tpu_sc as plsc

N_NODES = 10000
N_EDGES = 320000
D = 128

NUM_CORES = 2
NUM_SUBCORES = 16
NUM_TILES = NUM_CORES * NUM_SUBCORES

CHUNK = 128
CHUNKS_PER_TILE = 80
EDGES_PER_TILE = CHUNK * CHUNKS_PER_TILE
N_PAD_EDGES = EDGES_PER_TILE * NUM_TILES
TRASH_ROW = N_NODES
N_ACC = 10240
ROWS_PER_TILE_ZERO = N_ACC // NUM_SUBCORES
ROWS_PER_TILE_OUT = 624

_MM_BLOCK = 1000


def _stage1_body(x_ref, wlT_ref, bl_ref, wrT_ref, xl_ref, outr_ref):
    x = x_ref[...]
    xl_ref[...] = (
        jnp.dot(x, wlT_ref[...], preferred_element_type=jnp.float32)
        + bl_ref[...]
    )
    outr_ref[...] = jnp.dot(x, wrT_ref[...], preferred_element_type=jnp.float32)


def _stage1(x, wlT, bl2d, wrT):
    grid = (N_NODES // _MM_BLOCK,)
    return pl.pallas_call(
        _stage1_body,
        grid=grid,
        in_specs=[
            pl.BlockSpec((_MM_BLOCK, D), lambda i: (i, 0)),
            pl.BlockSpec((D, D), lambda i: (0, 0)),
            pl.BlockSpec((1, D), lambda i: (0, 0)),
            pl.BlockSpec((D, D), lambda i: (0, 0)),
        ],
        out_specs=[
            pl.BlockSpec((_MM_BLOCK, D), lambda i: (i, 0)),
            pl.BlockSpec((_MM_BLOCK, D), lambda i: (i, 0)),
        ],
        out_shape=[
            jax.ShapeDtypeStruct((N_NODES, D), jnp.float32),
            jax.ShapeDtypeStruct((N_NODES, D), jnp.float32),
        ],
    )(x, wlT, bl2d, wrT)


def _sc_body(
    xl_hbm, src_hbm, dst_hbm, out_hbm,
    srcA, dstA, srcB, dstB, rA, rB,
    acc_sh,
    gA, gB,
):
    cid = lax.axis_index("c")
    sid = lax.axis_index("s")
    tile = cid * NUM_SUBCORES + sid
    e0 = tile * EDGES_PER_TILE

    zeros16 = jnp.zeros((16,), jnp.float32)

    def zero_body(i, carry):
        r = i // (D // 16)
        c = i % (D // 16)
        rA[r, pl.ds(c * 16, 16)] = zeros16
        return carry

    lax.fori_loop(0, CHUNK * (D // 16), zero_body, 0)

    def zcopy_body(j, carry):
        pltpu.sync_copy(
            rA,
            acc_sh.at[pl.ds(sid * ROWS_PER_TILE_ZERO + j * CHUNK, CHUNK)],
        )
        return carry

    lax.fori_loop(0, ROWS_PER_TILE_ZERO // CHUNK, zcopy_body, 0)
    plsc.subcore_barrier()

    def body(i, carry):
        base = e0 + i * CHUNK
        pltpu.sync_copy(src_hbm.at[pl.ds(base, CHUNK)], srcA)
        pltpu.sync_copy(dst_hbm.at[pl.ds(base, CHUNK)], dstA)
        pltpu.async_copy(xl_hbm.at[srcA], rA, gA).wait()
        pltpu.sync_copy(rA, acc_sh.at[dstA], add=True)
        return carry

    lax.fori_loop(0, CHUNKS_PER_TILE, body, 0)
    plsc.subcore_barrier()

    row0 = sid * ROWS_PER_TILE_OUT
    pltpu.sync_copy(
        acc_sh.at[pl.ds(row0, ROWS_PER_TILE_OUT)],
        out_hbm.at[pl.ds(cid * N_NODES + row0, ROWS_PER_TILE_OUT)],
    )
    tail0 = NUM_SUBCORES * ROWS_PER_TILE_OUT
    tail = N_NODES - tail0

    @pl.when(sid == NUM_SUBCORES - 1)
    def _write_tail():
        pltpu.sync_copy(
            acc_sh.at[pl.ds(tail0, tail)],
            out_hbm.at[pl.ds(cid * N_NODES + tail0, tail)],
        )


_sc_stage = functools.partial(
    pl.kernel,
    out_type=jax.ShapeDtypeStruct((NUM_CORES * N_NODES, D), jnp.float32),
    mesh=plsc.VectorSubcoreMesh(core_axis_name="c", subcore_axis_name="s"),
    scratch_types=(
        [pltpu.VMEM((CHUNK,), jnp.int32) for _ in range(4)]
        + [pltpu.VMEM((CHUNK, D), jnp.float32) for _ in range(2)]
        + [pltpu.VMEM_SHARED((N_ACC, D), jnp.float32)]
        + [pltpu.SemaphoreType.DMA for _ in range(2)]
    ),
)(_sc_body)


def _stage3_body(p0_ref, p1_ref, outr_ref, out_ref):
    out_ref[...] = p0_ref[...] + p1_ref[...] + outr_ref[...]


def _stage3(p0, p1, outr):
    grid = (N_NODES // _MM_BLOCK,)
    spec = pl.BlockSpec((_MM_BLOCK, D), lambda i: (i, 0))
    return pl.pallas_call(
        _stage3_body,
        grid=grid,
        in_specs=[spec, spec, spec],
        out_specs=spec,
        out_shape=jax.ShapeDtypeStruct((N_NODES, D), jnp.float32),
    )(p0, p1, outr)


def kernel(x, edge_index, W_l, b_l, W_r):
    src = edge_index[0]
    dst = edge_index[1]
    pad = N_PAD_EDGES + CHUNK - N_EDGES
    src_pad = jnp.concatenate([src, jnp.zeros((pad,), jnp.int32)])
    dst_pad = jnp.concatenate([dst, jnp.full((pad,), TRASH_ROW, jnp.int32)])

    xl, outr = _stage1(x, W_l.T, b_l.reshape(1, D), W_r.T)
    parts = _sc_stage(xl, src_pad, dst_pad)
    return _stage3(parts[:N_NODES], parts[N_NODES:], outr)


# speedup vs baseline: 1.0008x; 1.0008x over previous
import functools

import jax
import jax.numpy as jnp
from jax import lax
from jax.experimental import pallas as pl
from jax.experimental.pallas import tpu as pltpu
from jax.experimental.pallas import tpu_sc as plsc

N_NODES = 10000
N_EDGES = 320000
D = 128

NUM_CORES = 2
NUM_SUBCORES = 16
NUM_TILES = NUM_CORES * NUM_SUBCORES

CHUNK = 128
CHUNKS_PER_TILE = 80
EDGES_PER_TILE = CHUNK * CHUNKS_PER_TILE
N_PAD_EDGES = EDGES_PER_TILE * NUM_TILES
TRASH_ROW = N_NODES
N_ACC = 10240
ROWS_PER_TILE_ZERO = N_ACC // NUM_SUBCORES
ROWS_PER_TILE_OUT = 624

_MM_BLOCK = 1000


def _stage1_body(x_ref, wlT_ref, bl_ref, wrT_ref, xl_ref, outr_ref):
    x = x_ref[...]
    xl_ref[...] = (
        jnp.dot(x, wlT_ref[...], preferred_element_type=jnp.float32)
        + bl_ref[...]
    )
    outr_ref[...] = jnp.dot(x, wrT_ref[...], preferred_element_type=jnp.float32)


def _stage1(x, wlT, bl2d, wrT):
    grid = (N_NODES // _MM_BLOCK,)
    return pl.pallas_call(
        _stage1_body,
        grid=grid,
        in_specs=[
            pl.BlockSpec((_MM_BLOCK, D), lambda i: (i, 0)),
            pl.BlockSpec((D, D), lambda i: (0, 0)),
            pl.BlockSpec((1, D), lambda i: (0, 0)),
            pl.BlockSpec((D, D), lambda i: (0, 0)),
        ],
        out_specs=[
            pl.BlockSpec((_MM_BLOCK, D), lambda i: (i, 0)),
            pl.BlockSpec((_MM_BLOCK, D), lambda i: (i, 0)),
        ],
        out_shape=[
            jax.ShapeDtypeStruct((N_NODES, D), jnp.float32),
            jax.ShapeDtypeStruct((N_NODES, D), jnp.float32),
        ],
    )(x, wlT, bl2d, wrT)


def _sc_body(
    xl_hbm, src_hbm, dst_hbm, out_hbm,
    srcA, dstA, rA,
    acc_sh,
    gA,
):
    cid = lax.axis_index("c")
    sid = lax.axis_index("s")
    tile = cid * NUM_SUBCORES + sid
    e0 = tile * EDGES_PER_TILE

    zeros16 = jnp.zeros((16,), jnp.float32)

    def zero_body(i, carry):
        r = i // (D // 16)
        c = i % (D // 16)
        rA[r, pl.ds(c * 16, 16)] = zeros16
        return carry

    lax.fori_loop(0, CHUNK * (D // 16), zero_body, 0)

    def zcopy_body(j, carry):
        pltpu.sync_copy(
            rA,
            acc_sh.at[pl.ds(sid * ROWS_PER_TILE_ZERO + j * CHUNK, CHUNK)],
        )
        return carry

    lax.fori_loop(0, ROWS_PER_TILE_ZERO // CHUNK, zcopy_body, 0)
    plsc.subcore_barrier()

    def body(i, carry):
        base = e0 + i * CHUNK
        pltpu.sync_copy(src_hbm.at[pl.ds(base, CHUNK)], srcA)
        pltpu.sync_copy(dst_hbm.at[pl.ds(base, CHUNK)], dstA)
        pltpu.async_copy(xl_hbm.at[srcA], rA, gA).wait()
        pltpu.sync_copy(rA, acc_sh.at[dstA], add=True)
        return carry

    lax.fori_loop(0, CHUNKS_PER_TILE, body, 0)
    plsc.subcore_barrier()

    row0 = sid * ROWS_PER_TILE_OUT
    pltpu.sync_copy(
        acc_sh.at[pl.ds(row0, ROWS_PER_TILE_OUT)],
        out_hbm.at[pl.ds(cid * N_NODES + row0, ROWS_PER_TILE_OUT)],
    )
    tail0 = NUM_SUBCORES * ROWS_PER_TILE_OUT
    tail = N_NODES - tail0

    @pl.when(sid == NUM_SUBCORES - 1)
    def _write_tail():
        pltpu.sync_copy(
            acc_sh.at[pl.ds(tail0, tail)],
            out_hbm.at[pl.ds(cid * N_NODES + tail0, tail)],
        )


_sc_stage = functools.partial(
    pl.kernel,
    out_type=jax.ShapeDtypeStruct((NUM_CORES * N_NODES, D), jnp.float32),
    mesh=plsc.VectorSubcoreMesh(core_axis_name="c", subcore_axis_name="s"),
    scratch_types=(
        [pltpu.VMEM((CHUNK,), jnp.int32) for _ in range(2)]
        + [pltpu.VMEM((CHUNK, D), jnp.float32)]
        + [pltpu.VMEM_SHARED((N_ACC, D), jnp.float32)]
        + [pltpu.SemaphoreType.DMA]
    ),
)(_sc_body)


def _stage3_body(p0_ref, p1_ref, outr_ref, out_ref):
    out_ref[...] = p0_ref[...] + p1_ref[...] + outr_ref[...]


def _stage3(p0, p1, outr):
    grid = (N_NODES // _MM_BLOCK,)
    spec = pl.BlockSpec((_MM_BLOCK, D), lambda i: (i, 0))
    return pl.pallas_call(
        _stage3_body,
        grid=grid,
        in_specs=[spec, spec, spec],
        out_specs=spec,
        out_shape=jax.ShapeDtypeStruct((N_NODES, D), jnp.float32),
    )(p0, p1, outr)


def kernel(x, edge_index, W_l, b_l, W_r):
    src = edge_index[0]
    dst = edge_index[1]
    pad = N_PAD_EDGES + CHUNK - N_EDGES
    src_pad = jnp.concatenate([src, jnp.zeros((pad,), jnp.int32)])
    dst_pad = jnp.concatenate([dst, jnp.full((pad,), TRASH_ROW, jnp.int32)])

    xl, outr = _stage1(x, W_l.T, b_l.reshape(1, D), W_r.T)
    parts = _sc_stage(xl, src_pad, dst_pad)
    return _stage3(parts[:N_NODES], parts[N_NODES:], outr)


# speedup vs baseline: 1.4505x; 1.4494x over previous
import functools

import jax
import jax.numpy as jnp
from jax import lax
from jax.experimental import pallas as pl
from jax.experimental.pallas import tpu as pltpu
from jax.experimental.pallas import tpu_sc as plsc

N_NODES = 10000
N_EDGES = 320000
D = 128

NUM_CORES = 2
NUM_SUBCORES = 16
NUM_TILES = NUM_CORES * NUM_SUBCORES

CHUNK = 128
CHUNKS_PER_TILE = 79
EDGES_PER_TILE = CHUNK * CHUNKS_PER_TILE
N_PAD_EDGES = EDGES_PER_TILE * NUM_TILES
REAL_PER_TILE = N_EDGES // NUM_TILES
PAD_PER_TILE = EDGES_PER_TILE - REAL_PER_TILE
TRASH_ROW = N_NODES
N_ACC = 10240
ROWS_PER_TILE_ZERO = N_ACC // NUM_SUBCORES
ROWS_PER_TILE_OUT = 624

_MM_BLOCK = 1000


def _stage1_body(x_ref, wlT_ref, bl_ref, wrT_ref, xl_ref, outr_ref):
    x = x_ref[...]
    xl_ref[...] = (
        jnp.dot(x, wlT_ref[...], preferred_element_type=jnp.float32)
        + bl_ref[...]
    )
    outr_ref[...] = jnp.dot(x, wrT_ref[...], preferred_element_type=jnp.float32)


def _stage1(x, wlT, bl2d, wrT):
    grid = (N_NODES // _MM_BLOCK,)
    return pl.pallas_call(
        _stage1_body,
        grid=grid,
        in_specs=[
            pl.BlockSpec((_MM_BLOCK, D), lambda i: (i, 0)),
            pl.BlockSpec((D, D), lambda i: (0, 0)),
            pl.BlockSpec((1, D), lambda i: (0, 0)),
            pl.BlockSpec((D, D), lambda i: (0, 0)),
        ],
        out_specs=[
            pl.BlockSpec((_MM_BLOCK, D), lambda i: (i, 0)),
            pl.BlockSpec((_MM_BLOCK, D), lambda i: (i, 0)),
        ],
        out_shape=[
            jax.ShapeDtypeStruct((N_NODES, D), jnp.float32),
            jax.ShapeDtypeStruct((N_NODES, D), jnp.float32),
        ],
    )(x, wlT, bl2d, wrT)


def _sc_body(
    xl_hbm, src_hbm, dst_hbm, out_hbm,
    srcA, dstA, rA,
    acc_sh,
    gA,
):
    cid = lax.axis_index("c")
    sid = lax.axis_index("s")
    tile = cid * NUM_SUBCORES + sid
    e0 = tile * EDGES_PER_TILE

    zeros16 = jnp.zeros((16,), jnp.float32)

    def zero_body(i, carry):
        r = i // (D // 16)
        c = i % (D // 16)
        rA[r, pl.ds(c * 16, 16)] = zeros16
        return carry

    lax.fori_loop(0, CHUNK * (D // 16), zero_body, 0)

    def zcopy_body(j, carry):
        pltpu.sync_copy(
            rA,
            acc_sh.at[pl.ds(sid * ROWS_PER_TILE_ZERO + j * CHUNK, CHUNK)],
        )
        return carry

    lax.fori_loop(0, ROWS_PER_TILE_ZERO // CHUNK, zcopy_body, 0)
    plsc.subcore_barrier()

    def body(i, carry):
        base = e0 + i * CHUNK
        pltpu.sync_copy(src_hbm.at[pl.ds(base, CHUNK)], srcA)
        pltpu.sync_copy(dst_hbm.at[pl.ds(base, CHUNK)], dstA)
        pltpu.async_copy(xl_hbm.at[srcA], rA, gA).wait()
        pltpu.sync_copy(rA, acc_sh.at[dstA], add=True)
        return carry

    lax.fori_loop(0, CHUNKS_PER_TILE, body, 0)
    plsc.subcore_barrier()

    row0 = sid * ROWS_PER_TILE_OUT
    pltpu.sync_copy(
        acc_sh.at[pl.ds(row0, ROWS_PER_TILE_OUT)],
        out_hbm.at[pl.ds(cid * N_NODES + row0, ROWS_PER_TILE_OUT)],
    )
    tail0 = NUM_SUBCORES * ROWS_PER_TILE_OUT
    tail = N_NODES - tail0

    @pl.when(sid == NUM_SUBCORES - 1)
    def _write_tail():
        pltpu.sync_copy(
            acc_sh.at[pl.ds(tail0, tail)],
            out_hbm.at[pl.ds(cid * N_NODES + tail0, tail)],
        )


_sc_stage = functools.partial(
    pl.kernel,
    out_type=jax.ShapeDtypeStruct((NUM_CORES * N_NODES, D), jnp.float32),
    mesh=plsc.VectorSubcoreMesh(core_axis_name="c", subcore_axis_name="s"),
    scratch_types=(
        [pltpu.VMEM((CHUNK,), jnp.int32) for _ in range(2)]
        + [pltpu.VMEM((CHUNK, D), jnp.float32)]
        + [pltpu.VMEM_SHARED((N_ACC, D), jnp.float32)]
        + [pltpu.SemaphoreType.DMA]
    ),
)(_sc_body)


def _stage3_body(p0_ref, p1_ref, outr_ref, out_ref):
    out_ref[...] = p0_ref[...] + p1_ref[...] + outr_ref[...]


def _stage3(p0, p1, outr):
    grid = (N_NODES // _MM_BLOCK,)
    spec = pl.BlockSpec((_MM_BLOCK, D), lambda i: (i, 0))
    return pl.pallas_call(
        _stage3_body,
        grid=grid,
        in_specs=[spec, spec, spec],
        out_specs=spec,
        out_shape=jax.ShapeDtypeStruct((N_NODES, D), jnp.float32),
    )(p0, p1, outr)


def kernel(x, edge_index, W_l, b_l, W_r):
    src2 = edge_index[0].reshape(NUM_TILES, REAL_PER_TILE)
    dst2 = edge_index[1].reshape(NUM_TILES, REAL_PER_TILE)
    pad_src = jnp.zeros((NUM_TILES, PAD_PER_TILE), jnp.int32)
    pad_dst = jnp.broadcast_to(
        TRASH_ROW + jnp.arange(PAD_PER_TILE, dtype=jnp.int32),
        (NUM_TILES, PAD_PER_TILE),
    )
    src_pad = jnp.concatenate([src2, pad_src], axis=1).reshape(-1)
    dst_pad = jnp.concatenate([dst2, pad_dst], axis=1).reshape(-1)

    xl, outr = _stage1(x, W_l.T, b_l.reshape(1, D), W_r.T)
    parts = _sc_stage(xl, src_pad, dst_pad)
    return _stage3(parts[:N_NODES], parts[N_NODES:], outr)


# speedup vs baseline: 2.0478x; 1.4118x over previous
import functools

import jax
import jax.numpy as jnp
from jax import lax
from jax.experimental import pallas as pl
from jax.experimental.pallas import tpu as pltpu
from jax.experimental.pallas import tpu_sc as plsc

N_NODES = 10000
N_EDGES = 320000
D = 128

NUM_CORES = 2
NUM_SUBCORES = 16
NUM_TILES = NUM_CORES * NUM_SUBCORES

CHUNK = 128
EDGES_PER_TILE = N_EDGES // NUM_TILES
FULL_CHUNKS = EDGES_PER_TILE // CHUNK
TAIL = EDGES_PER_TILE - FULL_CHUNKS * CHUNK
N_ACC = 10240
ROWS_PER_TILE_ZERO = N_ACC // NUM_SUBCORES
ROWS_PER_TILE_OUT = 624

_MM_BLOCK = 1000


def _stage1_body(x_ref, wlT_ref, bl_ref, wrT_ref, xl_ref, outr_ref):
    x = x_ref[...]
    xl_ref[...] = (
        jnp.dot(x, wlT_ref[...], preferred_element_type=jnp.float32)
        + bl_ref[...]
    )
    outr_ref[...] = jnp.dot(x, wrT_ref[...], preferred_element_type=jnp.float32)


def _stage1(x, wlT, bl2d, wrT):
    grid = (N_NODES // _MM_BLOCK,)
    return pl.pallas_call(
        _stage1_body,
        grid=grid,
        in_specs=[
            pl.BlockSpec((_MM_BLOCK, D), lambda i: (i, 0)),
            pl.BlockSpec((D, D), lambda i: (0, 0)),
            pl.BlockSpec((1, D), lambda i: (0, 0)),
            pl.BlockSpec((D, D), lambda i: (0, 0)),
        ],
        out_specs=[
            pl.BlockSpec((_MM_BLOCK, D), lambda i: (i, 0)),
            pl.BlockSpec((_MM_BLOCK, D), lambda i: (i, 0)),
        ],
        out_shape=[
            jax.ShapeDtypeStruct((N_NODES, D), jnp.float32),
            jax.ShapeDtypeStruct((N_NODES, D), jnp.float32),
        ],
    )(x, wlT, bl2d, wrT)


def _sc_body(
    xl_hbm, src_hbm, dst_hbm, out_hbm,
    srcA, dstA, srcT, dstT, rA,
    acc_sh,
    gA,
):
    cid = lax.axis_index("c")
    sid = lax.axis_index("s")
    tile = cid * NUM_SUBCORES + sid
    e0 = tile * EDGES_PER_TILE

    zeros16 = jnp.zeros((16,), jnp.float32)

    def zero_body(i, carry):
        r = i // (D // 16)
        c = i % (D // 16)
        rA[r, pl.ds(c * 16, 16)] = zeros16
        return carry

    lax.fori_loop(0, CHUNK * (D // 16), zero_body, 0)

    def zcopy_body(j, carry):
        pltpu.sync_copy(
            rA,
            acc_sh.at[pl.ds(sid * ROWS_PER_TILE_ZERO + j * CHUNK, CHUNK)],
        )
        return carry

    lax.fori_loop(0, ROWS_PER_TILE_ZERO // CHUNK, zcopy_body, 0)
    plsc.subcore_barrier()

    def body(i, carry):
        base = e0 + i * CHUNK
        pltpu.sync_copy(src_hbm.at[pl.ds(base, CHUNK)], srcA)
        pltpu.sync_copy(dst_hbm.at[pl.ds(base, CHUNK)], dstA)
        pltpu.async_copy(xl_hbm.at[srcA], rA, gA).wait()
        pltpu.sync_copy(rA, acc_sh.at[dstA], add=True)
        return carry

    lax.fori_loop(0, FULL_CHUNKS, body, 0)

    tbase = e0 + FULL_CHUNKS * CHUNK
    pltpu.sync_copy(src_hbm.at[pl.ds(tbase, TAIL)], srcT)
    pltpu.sync_copy(dst_hbm.at[pl.ds(tbase, TAIL)], dstT)
    pltpu.async_copy(xl_hbm.at[srcT], rA.at[pl.ds(0, TAIL)], gA).wait()
    pltpu.sync_copy(rA.at[pl.ds(0, TAIL)], acc_sh.at[dstT], add=True)
    plsc.subcore_barrier()

    row0 = sid * ROWS_PER_TILE_OUT
    pltpu.sync_copy(
        acc_sh.at[pl.ds(row0, ROWS_PER_TILE_OUT)],
        out_hbm.at[pl.ds(cid * N_NODES + row0, ROWS_PER_TILE_OUT)],
    )
    tail0 = NUM_SUBCORES * ROWS_PER_TILE_OUT
    tail = N_NODES - tail0

    @pl.when(sid == NUM_SUBCORES - 1)
    def _write_tail():
        pltpu.sync_copy(
            acc_sh.at[pl.ds(tail0, tail)],
            out_hbm.at[pl.ds(cid * N_NODES + tail0, tail)],
        )


_sc_stage = functools.partial(
    pl.kernel,
    out_type=jax.ShapeDtypeStruct((NUM_CORES * N_NODES, D), jnp.float32),
    mesh=plsc.VectorSubcoreMesh(core_axis_name="c", subcore_axis_name="s"),
    scratch_types=(
        [pltpu.VMEM((CHUNK,), jnp.int32) for _ in range(2)]
        + [pltpu.VMEM((TAIL,), jnp.int32) for _ in range(2)]
        + [pltpu.VMEM((CHUNK, D), jnp.float32)]
        + [pltpu.VMEM_SHARED((N_ACC, D), jnp.float32)]
        + [pltpu.SemaphoreType.DMA]
    ),
)(_sc_body)


def _stage3_body(p0_ref, p1_ref, outr_ref, out_ref):
    out_ref[...] = p0_ref[...] + p1_ref[...] + outr_ref[...]


def _stage3(p0, p1, outr):
    grid = (N_NODES // _MM_BLOCK,)
    spec = pl.BlockSpec((_MM_BLOCK, D), lambda i: (i, 0))
    return pl.pallas_call(
        _stage3_body,
        grid=grid,
        in_specs=[spec, spec, spec],
        out_specs=spec,
        out_shape=jax.ShapeDtypeStruct((N_NODES, D), jnp.float32),
    )(p0, p1, outr)


def kernel(x, edge_index, W_l, b_l, W_r):
    xl, outr = _stage1(x, W_l.T, b_l.reshape(1, D), W_r.T)
    parts = _sc_stage(xl, edge_index[0], edge_index[1])
    return _stage3(parts[:N_NODES], parts[N_NODES:], outr)


# speedup vs baseline: 2.3361x; 1.1408x over previous
import functools

import jax
import jax.numpy as jnp
from jax import lax
from jax.experimental import pallas as pl
from jax.experimental.pallas import tpu as pltpu
from jax.experimental.pallas import tpu_sc as plsc

N_NODES = 10000
N_EDGES = 320000
D = 128

NUM_CORES = 2
NUM_SUBCORES = 16
NUM_TILES = NUM_CORES * NUM_SUBCORES

CHUNK = 128
EDGES_PER_TILE = N_EDGES // NUM_TILES
FULL_CHUNKS = EDGES_PER_TILE // CHUNK
TAIL = EDGES_PER_TILE - FULL_CHUNKS * CHUNK
N_ACC = 10240
ROWS_PER_TILE_ZERO = N_ACC // NUM_SUBCORES
ROWS_PER_TILE_OUT = 624

_MM_BLOCK = 1000


def _stage1_body(x_ref, wlT_ref, bl_ref, xl_ref):
    xl_ref[...] = (
        jnp.dot(x_ref[...], wlT_ref[...], preferred_element_type=jnp.float32)
        + bl_ref[...]
    )


def _stage1(x, wlT, bl2d):
    grid = (N_NODES // _MM_BLOCK,)
    return pl.pallas_call(
        _stage1_body,
        grid=grid,
        in_specs=[
            pl.BlockSpec((_MM_BLOCK, D), lambda i: (i, 0)),
            pl.BlockSpec((D, D), lambda i: (0, 0)),
            pl.BlockSpec((1, D), lambda i: (0, 0)),
        ],
        out_specs=pl.BlockSpec((_MM_BLOCK, D), lambda i: (i, 0)),
        out_shape=jax.ShapeDtypeStruct((N_NODES, D), jnp.float32),
    )(x, wlT, bl2d)


def _sc_body(
    xl_hbm, src_hbm, dst_hbm, out_hbm,
    srcA, dstA, srcT, dstT, rA,
    acc_sh,
    gA,
):
    cid = lax.axis_index("c")
    sid = lax.axis_index("s")
    tile = cid * NUM_SUBCORES + sid
    e0 = tile * EDGES_PER_TILE

    zeros16 = jnp.zeros((16,), jnp.float32)

    def zero_body(i, carry):
        r = i // (D // 16)
        c = i % (D // 16)
        rA[r, pl.ds(c * 16, 16)] = zeros16
        return carry

    lax.fori_loop(0, CHUNK * (D // 16), zero_body, 0)

    def zcopy_body(j, carry):
        pltpu.sync_copy(
            rA,
            acc_sh.at[pl.ds(sid * ROWS_PER_TILE_ZERO + j * CHUNK, CHUNK)],
        )
        return carry

    lax.fori_loop(0, ROWS_PER_TILE_ZERO // CHUNK, zcopy_body, 0)
    plsc.subcore_barrier()

    def body(i, carry):
        base = e0 + i * CHUNK
        pltpu.sync_copy(src_hbm.at[pl.ds(base, CHUNK)], srcA)
        g = pltpu.async_copy(xl_hbm.at[srcA], rA, gA)
        pltpu.sync_copy(dst_hbm.at[pl.ds(base, CHUNK)], dstA)
        g.wait()
        pltpu.sync_copy(rA, acc_sh.at[dstA], add=True)
        return carry

    lax.fori_loop(0, FULL_CHUNKS, body, 0)

    tbase = e0 + FULL_CHUNKS * CHUNK
    pltpu.sync_copy(src_hbm.at[pl.ds(tbase, TAIL)], srcT)
    g = pltpu.async_copy(xl_hbm.at[srcT], rA.at[pl.ds(0, TAIL)], gA)
    pltpu.sync_copy(dst_hbm.at[pl.ds(tbase, TAIL)], dstT)
    g.wait()
    pltpu.sync_copy(rA.at[pl.ds(0, TAIL)], acc_sh.at[dstT], add=True)
    plsc.subcore_barrier()

    row0 = sid * ROWS_PER_TILE_OUT
    pltpu.sync_copy(
        acc_sh.at[pl.ds(row0, ROWS_PER_TILE_OUT)],
        out_hbm.at[pl.ds(cid * N_NODES + row0, ROWS_PER_TILE_OUT)],
    )
    tail0 = NUM_SUBCORES * ROWS_PER_TILE_OUT
    tail = N_NODES - tail0

    @pl.when(sid == NUM_SUBCORES - 1)
    def _write_tail():
        pltpu.sync_copy(
            acc_sh.at[pl.ds(tail0, tail)],
            out_hbm.at[pl.ds(cid * N_NODES + tail0, tail)],
        )


_sc_stage = functools.partial(
    pl.kernel,
    out_type=jax.ShapeDtypeStruct((NUM_CORES * N_NODES, D), jnp.float32),
    mesh=plsc.VectorSubcoreMesh(core_axis_name="c", subcore_axis_name="s"),
    scratch_types=(
        [pltpu.VMEM((CHUNK,), jnp.int32) for _ in range(2)]
        + [pltpu.VMEM((TAIL,), jnp.int32) for _ in range(2)]
        + [pltpu.VMEM((CHUNK, D), jnp.float32)]
        + [pltpu.VMEM_SHARED((N_ACC, D), jnp.float32)]
        + [pltpu.SemaphoreType.DMA]
    ),
)(_sc_body)


def _stage3_body(p0_ref, p1_ref, x_ref, wrT_ref, out_ref):
    out_ref[...] = (
        p0_ref[...]
        + p1_ref[...]
        + jnp.dot(x_ref[...], wrT_ref[...], preferred_element_type=jnp.float32)
    )


def _stage3(p0, p1, x, wrT):
    grid = (N_NODES // _MM_BLOCK,)
    spec = pl.BlockSpec((_MM_BLOCK, D), lambda i: (i, 0))
    return pl.pallas_call(
        _stage3_body,
        grid=grid,
        in_specs=[spec, spec, spec, pl.BlockSpec((D, D), lambda i: (0, 0))],
        out_specs=spec,
        out_shape=jax.ShapeDtypeStruct((N_NODES, D), jnp.float32),
    )(p0, p1, x, wrT)


def kernel(x, edge_index, W_l, b_l, W_r):
    xl = _stage1(x, W_l.T, b_l.reshape(1, D))
    parts = _sc_stage(xl, edge_index[0], edge_index[1])
    return _stage3(parts[:N_NODES], parts[N_NODES:], x, W_r.T)
